# baseline (speedup 1.0000x reference)
import jax
import jax.numpy as jnp
from jax.experimental import pallas as pl
from jax.experimental.pallas import tpu as pltpu

_TEMPERATURE = 2.0
_BLOCK_T = 1024
_OUT_GROUP = 8


def _router_block(x_ref, wt_ref, b_ref, out_ref):
    i = pl.program_id(0)
    logits = jnp.dot(x_ref[...], wt_ref[...], preferred_element_type=jnp.float32)
    logits = (logits + b_ref[...]) * (1.0 / _TEMPERATURE)
    m = jnp.max(logits, axis=-1, keepdims=True)
    e = jnp.exp(logits - m)
    probs = e / jnp.sum(e, axis=-1, keepdims=True)
    out_ref[pl.ds(i * _BLOCK_T, _BLOCK_T), :] = probs


def kernel(x, W, b):
    tokens, d_model = x.shape
    num_experts = W.shape[0]
    wt = W.T
    b2 = b.reshape(1, num_experts)
    bt = _BLOCK_T
    return pl.pallas_call(
        _router_block,
        grid=(tokens // bt,),
        in_specs=[
            pl.BlockSpec((bt, d_model), lambda i: (i, 0)),
            pl.BlockSpec((d_model, num_experts), lambda i: (0, 0)),
            pl.BlockSpec((1, num_experts), lambda i: (0, 0)),
        ],
        out_specs=pl.BlockSpec((tokens, num_experts), lambda i: (0, 0)),
        out_shape=jax.ShapeDtypeStruct((tokens, num_experts), jnp.float32),
    )(x, wt, b2)


# speedup vs baseline: 1.0103x; 1.0103x over previous
import jax
import jax.numpy as jnp
from jax.experimental import pallas as pl
from jax.experimental.pallas import tpu as pltpu

_TEMPERATURE = 2.0
_BLOCK_T = 1024


def _router_block(x_ref, wt_ref, b_ref, out_hbm, obuf, osems):
    i = pl.program_id(0)
    n = pl.num_programs(0)

    def out_copy(j, oslot):
        return pltpu.make_async_copy(
            obuf.at[oslot],
            out_hbm.at[pl.ds(j * _BLOCK_T, _BLOCK_T), :],
            osems.at[oslot],
        )

    logits = jnp.dot(x_ref[...], wt_ref[...], preferred_element_type=jnp.float32)
    logits = (logits + b_ref[...]) * (1.0 / _TEMPERATURE)
    m = jnp.max(logits, axis=-1, keepdims=True)
    e = jnp.exp(logits - m)
    probs = e / jnp.sum(e, axis=-1, keepdims=True)

    oslot = jax.lax.rem(i, 2)

    @pl.when(i >= 2)
    def _():
        out_copy(i - 2, oslot).wait()

    obuf[oslot] = probs
    out_copy(i, oslot).start()

    @pl.when(i == n - 1)
    def _():
        out_copy(n - 2, jax.lax.rem(n - 2, 2)).wait()
        out_copy(n - 1, jax.lax.rem(n - 1, 2)).wait()


def kernel(x, W, b):
    tokens, d_model = x.shape
    num_experts = W.shape[0]
    wt = W.T
    b2 = b.reshape(1, num_experts)
    bt = _BLOCK_T
    return pl.pallas_call(
        _router_block,
        grid=(tokens // bt,),
        in_specs=[
            pl.BlockSpec((bt, d_model), lambda i: (i, 0)),
            pl.BlockSpec((d_model, num_experts), lambda i: (0, 0)),
            pl.BlockSpec((1, num_experts), lambda i: (0, 0)),
        ],
        out_specs=pl.BlockSpec(memory_space=pl.ANY),
        out_shape=jax.ShapeDtypeStruct((tokens, num_experts), jnp.float32),
        scratch_shapes=[
            pltpu.VMEM((2, bt, num_experts), jnp.float32),
            pltpu.SemaphoreType.DMA((2,)),
        ],
    )(x, wt, b2)
